# Initial kernel scaffold; baseline (speedup 1.0000x reference)
import jax
import jax.numpy as jnp
from jax.experimental import pallas as pl


def kernel(source_nodes, destination_nodes, edge_times, edge_idxs, node_feat, edge_feat, memory, time_w, time_b, nbr_idx, nbr_eidx, nbr_ts, Wq, Wk, Wv, Wo, Wm1, Wm2):
    raise NotImplementedError("write your pallas kernel here")


# speedup vs baseline: 2.4983x; 2.4983x over previous
import functools
import math

import jax
import jax.numpy as jnp
from jax import lax
from jax.experimental import pallas as pl
from jax.experimental.pallas import tpu as pltpu
from jax.experimental.pallas import tpu_sc as plsc

N_NODES = 10000
D = 128
K = 10
B2 = 2048
N1 = B2 * K
N2 = N1 * K
NE = N2 + N1


def _add_body(m_ref, f_ref, o_ref):
    o_ref[...] = m_ref[...] + f_ref[...]


def _feats_all(memory, node_feat):
    n = memory.shape[0]
    blk = 1000
    spec = pl.BlockSpec((blk, D), lambda b: (b, 0))
    return pl.pallas_call(
        _add_body,
        grid=(n // blk,),
        in_specs=[spec, spec],
        out_specs=spec,
        out_shape=jax.ShapeDtypeStruct((n, D), jnp.float32),
    )(memory, node_feat)


def _sc_info():
    info = plsc.get_sparse_core_info()
    return info.num_cores, info.num_subcores


def _gather_stream(tab, idxv, nch, chunk, buf0, buf1, s0, s1, sw, out, base):
    def pair(p, c):
        j0 = p * 2
        g0 = pltpu.async_copy(tab.at[idxv.at[j0]], buf0, s0)
        g1 = pltpu.async_copy(tab.at[idxv.at[j0 + 1]], buf1, s1)
        g0.wait()
        w0 = pltpu.async_copy(buf0, out.at[pl.ds(base + j0 * chunk, chunk)],
                              sw)
        g1.wait()
        w1 = pltpu.async_copy(
            buf1, out.at[pl.ds(base + (j0 + 1) * chunk, chunk)], sw)
        w0.wait()
        w1.wait()
        return c

    lax.fori_loop(0, nch // 2, pair, 0)
    if nch % 2:
        j = nch - 1
        pltpu.async_copy(tab.at[idxv.at[j]], buf0, s0).wait()
        pltpu.sync_copy(buf0, out.at[pl.ds(base + j * chunk, chunk)])


def _sc_gather(table, idx, chunk):
    v, d = table.shape
    n = idx.shape[0]
    nc, ns = _sc_info()
    nw = nc * ns
    nt = n // nw
    nch = nt // chunk
    idx2d = idx.reshape(nw * nch, chunk)
    mesh = plsc.VectorSubcoreMesh(core_axis_name="c", subcore_axis_name="s")

    @functools.partial(
        pl.kernel,
        mesh=mesh,
        out_type=jax.ShapeDtypeStruct((n, d), table.dtype),
        compiler_params=pltpu.CompilerParams(use_tc_tiling_on_sc=False),
        scratch_types=[
            pltpu.VMEM((nch, chunk), jnp.int32),
            pltpu.VMEM((chunk, d), table.dtype),
            pltpu.VMEM((chunk, d), table.dtype),
            pltpu.SemaphoreType.DMA,
            pltpu.SemaphoreType.DMA,
            pltpu.SemaphoreType.DMA,
        ],
    )
    def k(tab, idxr, out, idx_v, buf0, buf1, s0, s1, sw):
        wid = lax.axis_index("s") * nc + lax.axis_index("c")
        pltpu.sync_copy(idxr.at[pl.ds(wid * nch, nch)], idx_v)
        _gather_stream(tab, idx_v, nch, chunk, buf0, buf1, s0, s1, sw,
                       out, wid * nt)

    return k(table, idx2d)


def _sc_gather_feats(feats, ef16, idx_a, idx_b, idx_c, idx_e):
    nc, ns = _sc_info()
    nw = nc * ns
    ca, cb, cc, ce = 64, 128, 128, 80
    na, nb, nch, nche = 1, (N1 // nw) // cb, (N2 // nw) // cc, (NE // nw) // ce
    idx_a2 = idx_a.reshape(nw * na, ca)
    idx_b2 = idx_b.reshape(nw * nb, cb)
    idx_c2 = idx_c.reshape(nw * nch, cc)
    idx_e2 = idx_e.reshape(nw * nche, ce)
    mesh = plsc.VectorSubcoreMesh(core_axis_name="c", subcore_axis_name="s")

    @functools.partial(
        pl.kernel,
        mesh=mesh,
        out_type=(
            jax.ShapeDtypeStruct((B2, D), jnp.float32),
            jax.ShapeDtypeStruct((N1, D), jnp.float32),
            jax.ShapeDtypeStruct((N2, D), jnp.float32),
            jax.ShapeDtypeStruct((NE, 16), jnp.float32),
        ),
        compiler_params=pltpu.CompilerParams(use_tc_tiling_on_sc=False),
        scratch_types=[
            pltpu.VMEM((na, ca), jnp.int32),
            pltpu.VMEM((nb, cb), jnp.int32),
            pltpu.VMEM((nch, cc), jnp.int32),
            pltpu.VMEM((nche, ce), jnp.int32),
            pltpu.VMEM((ca, D), jnp.float32),
            pltpu.VMEM((cb, D), jnp.float32),
            pltpu.VMEM((cb, D), jnp.float32),
            pltpu.VMEM((ce, 16), jnp.float32),
            pltpu.VMEM((ce, 16), jnp.float32),
            pltpu.SemaphoreType.DMA,
            pltpu.SemaphoreType.DMA,
            pltpu.SemaphoreType.DMA,
        ],
    )
    def k(ftab, etab, ia, ib, ic, ie, oa, ob, oc, oe,
          iva, ivb, ivc, ive, bufa, buf0, buf1, ebuf0, ebuf1, s0, s1, sw):
        wid = lax.axis_index("s") * nc + lax.axis_index("c")

        pltpu.sync_copy(ia.at[pl.ds(wid * na, na)], iva)
        pltpu.async_copy(ftab.at[iva.at[0]], bufa, s0).wait()
        pltpu.sync_copy(bufa, oa.at[pl.ds(wid * ca, ca)])

        pltpu.sync_copy(ib.at[pl.ds(wid * nb, nb)], ivb)
        _gather_stream(ftab, ivb, nb, cb, buf0, buf1, s0, s1, sw,
                       ob, wid * (nb * cb))

        pltpu.sync_copy(ic.at[pl.ds(wid * nch, nch)], ivc)
        _gather_stream(ftab, ivc, nch, cc, buf0, buf1, s0, s1, sw,
                       oc, wid * (nch * cc))

        pltpu.sync_copy(ie.at[pl.ds(wid * nche, nche)], ive)
        _gather_stream(etab, ive, nche, ce, ebuf0, ebuf1, s0, s1, sw,
                       oe, wid * (nche * ce))

    return k(feats, ef16, idx_a2, idx_b2, idx_c2, idx_e2)


_INV_2PI = 0.15915494309189535
_COS_COEF = (0.9999999890590229, -19.73920449945394, 64.93911745989782,
             -85.45013953091483, 60.167630951137724, -25.967599248953547,
             6.528658161724017)


def _cos_2pi(y):
    r = y - jnp.round(y)
    u = r * r
    acc = jnp.full_like(u, _COS_COEF[6])
    for k in (5, 4, 3, 2, 1, 0):
        acc = acc * u + _COS_COEF[k]
    return acc


def _attn_body(tsv_ref, nbts_ref, ef_ref, cf_ref, nbf_ref, tw_ref, tb_ref,
               wqa, wqb, wka, wke, wkt, wva, wve, wvt, wo, wm1a, wm1b, wm2,
               o_ref, vbuf):
    f32 = jnp.float32
    feats = cf_ref[...]
    tw = tw_ref[...]
    tb = tb_ref[...]
    tsv = tsv_ref[...]
    twc = tw * _INV_2PI
    tbc = tb * _INV_2PI
    te0 = _cos_2pi(tbc)
    q = (jnp.dot(feats, wqa[...], preferred_element_type=f32)
         + jnp.dot(te0, wqb[...], preferred_element_type=f32))
    s0l, s1l = [], []
    for j in range(K):
        nbj = nbf_ref[:, j * D:(j + 1) * D]
        te = _cos_2pi((tsv - nbts_ref[:, j:j + 1]) * twc + tbc)
        efj = ef_ref[:, j:j + 1]
        kj = (jnp.dot(nbj, wka[...], preferred_element_type=f32)
              + jnp.dot(te, wkt[...], preferred_element_type=f32)
              + efj * wke[...])
        vj = (jnp.dot(nbj, wva[...], preferred_element_type=f32)
              + jnp.dot(te, wvt[...], preferred_element_type=f32)
              + efj * wve[...])
        vbuf[:, j * D:(j + 1) * D] = vj
        qk = q * kj
        s0l.append(jnp.sum(qk[:, :D // 2], axis=1, keepdims=True))
        s1l.append(jnp.sum(qk[:, D // 2:], axis=1, keepdims=True))
    inv = 1.0 / math.sqrt(D // 2)
    s0 = jnp.concatenate(s0l, axis=1) * inv
    s1 = jnp.concatenate(s1l, axis=1) * inv
    e0 = jnp.exp(s0 - jnp.max(s0, axis=1, keepdims=True))
    e1 = jnp.exp(s1 - jnp.max(s1, axis=1, keepdims=True))
    a0 = e0 / jnp.sum(e0, axis=1, keepdims=True)
    a1 = e1 / jnp.sum(e1, axis=1, keepdims=True)
    out0 = a0[:, 0:1] * vbuf[:, 0:D // 2]
    out1 = a1[:, 0:1] * vbuf[:, D // 2:D]
    for j in range(1, K):
        out0 = out0 + a0[:, j:j + 1] * vbuf[:, j * D:j * D + D // 2]
        out1 = out1 + a1[:, j:j + 1] * vbuf[:, j * D + D // 2:(j + 1) * D]
    out = jnp.concatenate([out0, out1], axis=1)
    ao = jnp.dot(out, wo[...], preferred_element_type=f32)
    hid = jnp.maximum(jnp.dot(ao, wm1a[...], preferred_element_type=f32)
                      + jnp.dot(feats, wm1b[...], preferred_element_type=f32),
                      0.0)
    o_ref[...] = jnp.dot(hid, wm2[...], preferred_element_type=f32)


def _attn_layer(tsv, nbts, ef, cf, nbf, tw2, tb2, w, blk):
    n = cf.shape[0]
    row = lambda width: pl.BlockSpec((blk, width), lambda b: (b, 0))
    full = lambda a: pl.BlockSpec(a.shape, lambda b: (0, 0))
    in_specs = [row(1), row(16), row(16), row(D), row(K * D),
                full(tw2), full(tb2)] + [full(x) for x in w]
    return pl.pallas_call(
        _attn_body,
        grid=(n // blk,),
        in_specs=in_specs,
        out_specs=row(D),
        out_shape=jax.ShapeDtypeStruct((n, D), jnp.float32),
        scratch_shapes=[pltpu.VMEM((blk, K * D), jnp.float32)],
    )(tsv, nbts, ef, cf, nbf, tw2, tb2, *w)


def _layer_weights(Wq, Wk, Wv, Wo, Wm1, Wm2, l):
    return (Wq[l, :D], Wq[l, D:],
            Wk[l, :D], Wk[l, D:D + 1], Wk[l, D + 1:],
            Wv[l, :D], Wv[l, D:D + 1], Wv[l, D + 1:],
            Wo[l], Wm1[l, :D], Wm1[l, D:], Wm2[l])


def kernel(source_nodes, destination_nodes, edge_times, edge_idxs, node_feat,
           edge_feat, memory, time_w, time_b, nbr_idx, nbr_eidx, nbr_ts,
           Wq, Wk, Wv, Wo, Wm1, Wm2):
    i32 = jnp.int32
    nodes = jnp.concatenate([source_nodes, destination_nodes]).astype(i32)
    ts2 = jnp.concatenate([edge_times, edge_times])

    nbrtab = jnp.concatenate(
        [nbr_idx.astype(i32), nbr_eidx.astype(i32),
         lax.bitcast_convert_type(nbr_ts, i32),
         jnp.zeros((N_NODES, 2), i32)], axis=1)

    feats = _feats_all(memory, node_feat)

    ta = _sc_gather(nbrtab, nodes, 64)
    c1idx = ta[:, :K].reshape(-1)
    eidx2 = ta[:, K:2 * K].reshape(-1)
    nbts2 = jnp.pad(lax.bitcast_convert_type(ta[:, 2 * K:3 * K], jnp.float32),
                    ((0, 0), (0, 16 - K)))

    tb_rows = _sc_gather(nbrtab, c1idx, 64)
    idx2 = tb_rows[:, :K].reshape(-1)
    eidx1 = tb_rows[:, K:2 * K].reshape(-1)
    nbts1 = jnp.pad(lax.bitcast_convert_type(tb_rows[:, 2 * K:3 * K],
                                             jnp.float32),
                    ((0, 0), (0, 16 - K)))

    ef_idx = jnp.concatenate([eidx1, eidx2])
    ef16 = jnp.pad(edge_feat, ((0, 0), (0, 15)))
    cf2, cf1, nbf, efg = _sc_gather_feats(feats, ef16, nodes, c1idx,
                                          idx2, ef_idx)
    ef1 = jnp.pad(efg[:N2, :1].reshape(N1, K), ((0, 0), (0, 16 - K)))
    ef2 = jnp.pad(efg[N2:, :1].reshape(B2, K), ((0, 0), (0, 16 - K)))

    tw2 = time_w.reshape(1, D)
    tb2 = time_b.reshape(1, D)
    tsr1 = jnp.repeat(ts2, K).reshape(N1, 1)
    w0 = _layer_weights(Wq, Wk, Wv, Wo, Wm1, Wm2, 0)
    w1 = _layer_weights(Wq, Wk, Wv, Wo, Wm1, Wm2, 1)

    l1 = _attn_layer(tsr1, nbts1, ef1, cf1, nbf.reshape(N1, K * D),
                     tw2, tb2, w0, 1024)
    out = _attn_layer(ts2.reshape(B2, 1), nbts2, ef2, cf2,
                      l1.reshape(B2, K * D), tw2, tb2, w1, 1024)
    nS = source_nodes.shape[0]
    return (out[:nS], out[nS:])


# speedup vs baseline: 2.7390x; 1.0963x over previous
import functools
import math

import jax
import jax.numpy as jnp
from jax import lax
from jax.experimental import pallas as pl
from jax.experimental.pallas import tpu as pltpu
from jax.experimental.pallas import tpu_sc as plsc

N_NODES = 10000
D = 128
K = 10
B2 = 2048
N1 = B2 * K
N2 = N1 * K
NE = N2 + N1


def _add_body(m_ref, f_ref, o_ref):
    o_ref[...] = m_ref[...] + f_ref[...]


def _feats_all(memory, node_feat):
    n = memory.shape[0]
    blk = 1000
    spec = pl.BlockSpec((blk, D), lambda b: (b, 0))
    return pl.pallas_call(
        _add_body,
        grid=(n // blk,),
        in_specs=[spec, spec],
        out_specs=spec,
        out_shape=jax.ShapeDtypeStruct((n, D), jnp.float32),
    )(memory, node_feat)


def _sc_info():
    info = plsc.get_sparse_core_info()
    return info.num_cores, info.num_subcores


def _gather_stream(tab, idxv, nch, chunk, buf0, buf1, s0, s1, sw, out, base):
    def pair(p, c):
        j0 = p * 2
        g0 = pltpu.async_copy(tab.at[idxv.at[j0]], buf0, s0)
        g1 = pltpu.async_copy(tab.at[idxv.at[j0 + 1]], buf1, s1)
        g0.wait()
        w0 = pltpu.async_copy(buf0, out.at[pl.ds(base + j0 * chunk, chunk)],
                              sw)
        g1.wait()
        w1 = pltpu.async_copy(
            buf1, out.at[pl.ds(base + (j0 + 1) * chunk, chunk)], sw)
        w0.wait()
        w1.wait()
        return c

    lax.fori_loop(0, nch // 2, pair, 0)
    if nch % 2:
        j = nch - 1
        pltpu.async_copy(tab.at[idxv.at[j]], buf0, s0).wait()
        pltpu.sync_copy(buf0, out.at[pl.ds(base + j * chunk, chunk)])


def _sc_gather(table, idx, chunk):
    v, d = table.shape
    n = idx.shape[0]
    nc, ns = _sc_info()
    nw = nc * ns
    nt = n // nw
    nch = nt // chunk
    idx2d = idx.reshape(nw * nch, chunk)
    mesh = plsc.VectorSubcoreMesh(core_axis_name="c", subcore_axis_name="s")

    @functools.partial(
        pl.kernel,
        mesh=mesh,
        out_type=jax.ShapeDtypeStruct((n, d), table.dtype),
        compiler_params=pltpu.CompilerParams(use_tc_tiling_on_sc=False),
        scratch_types=[
            pltpu.VMEM((nch, chunk), jnp.int32),
            pltpu.VMEM((chunk, d), table.dtype),
            pltpu.VMEM((chunk, d), table.dtype),
            pltpu.SemaphoreType.DMA,
            pltpu.SemaphoreType.DMA,
            pltpu.SemaphoreType.DMA,
        ],
    )
    def k(tab, idxr, out, idx_v, buf0, buf1, s0, s1, sw):
        wid = lax.axis_index("s") * nc + lax.axis_index("c")
        pltpu.sync_copy(idxr.at[pl.ds(wid * nch, nch)], idx_v)
        _gather_stream(tab, idx_v, nch, chunk, buf0, buf1, s0, s1, sw,
                       out, wid * nt)

    return k(table, idx2d)


def _sc_gather_feats(feats, ef16, idx_a, idx_b, idx_c, idx_e):
    nc, ns = _sc_info()
    nw = nc * ns
    ca, cb, cc, ce = 64, 128, 128, 80
    na, nb, nch, nche = 1, (N1 // nw) // cb, (N2 // nw) // cc, (NE // nw) // ce
    idx_a2 = idx_a.reshape(nw * na, ca)
    idx_b2 = idx_b.reshape(nw * nb, cb)
    idx_c2 = idx_c.reshape(nw * nch, cc)
    idx_e2 = idx_e.reshape(nw * nche, ce)
    mesh = plsc.VectorSubcoreMesh(core_axis_name="c", subcore_axis_name="s")

    @functools.partial(
        pl.kernel,
        mesh=mesh,
        out_type=(
            jax.ShapeDtypeStruct((B2, D), jnp.float32),
            jax.ShapeDtypeStruct((N1, D), jnp.float32),
            jax.ShapeDtypeStruct((N2, D), jnp.float32),
            jax.ShapeDtypeStruct((NE, 16), jnp.float32),
        ),
        compiler_params=pltpu.CompilerParams(use_tc_tiling_on_sc=False),
        scratch_types=[
            pltpu.VMEM((na, ca), jnp.int32),
            pltpu.VMEM((nb, cb), jnp.int32),
            pltpu.VMEM((nch, cc), jnp.int32),
            pltpu.VMEM((nche, ce), jnp.int32),
            pltpu.VMEM((ca, D), jnp.float32),
            pltpu.VMEM((cb, D), jnp.float32),
            pltpu.VMEM((cb, D), jnp.float32),
            pltpu.VMEM((ce, 16), jnp.float32),
            pltpu.VMEM((ce, 16), jnp.float32),
            pltpu.SemaphoreType.DMA,
            pltpu.SemaphoreType.DMA,
            pltpu.SemaphoreType.DMA,
        ],
    )
    def k(ftab, etab, ia, ib, ic, ie, oa, ob, oc, oe,
          iva, ivb, ivc, ive, bufa, buf0, buf1, ebuf0, ebuf1, s0, s1, sw):
        wid = lax.axis_index("s") * nc + lax.axis_index("c")

        pltpu.sync_copy(ia.at[pl.ds(wid * na, na)], iva)
        pltpu.async_copy(ftab.at[iva.at[0]], bufa, s0).wait()
        pltpu.sync_copy(bufa, oa.at[pl.ds(wid * ca, ca)])

        pltpu.sync_copy(ib.at[pl.ds(wid * nb, nb)], ivb)
        _gather_stream(ftab, ivb, nb, cb, buf0, buf1, s0, s1, sw,
                       ob, wid * (nb * cb))

        pltpu.sync_copy(ic.at[pl.ds(wid * nch, nch)], ivc)
        _gather_stream(ftab, ivc, nch, cc, buf0, buf1, s0, s1, sw,
                       oc, wid * (nch * cc))

        pltpu.sync_copy(ie.at[pl.ds(wid * nche, nche)], ive)
        _gather_stream(etab, ive, nche, ce, ebuf0, ebuf1, s0, s1, sw,
                       oe, wid * (nche * ce))

    return k(feats, ef16, idx_a2, idx_b2, idx_c2, idx_e2)


_INV_2PI = 0.15915494309189535
_COS_COEF = (0.9999999890590229, -19.73920449945394, 64.93911745989782,
             -85.45013953091483, 60.167630951137724, -25.967599248953547,
             6.528658161724017)


def _cos_2pi(y):
    r = y - jnp.round(y)
    u = r * r
    acc = jnp.full_like(u, _COS_COEF[6])
    for k in (5, 4, 3, 2, 1, 0):
        acc = acc * u + _COS_COEF[k]
    return acc


def _attn_body(tsv_ref, nbts_ref, ef_ref, cf_ref, nbf_ref, tw_ref, tb_ref,
               sel_ref,
               wqa, wqb, wka, wke, wkt, wva, wve, wvt, wo, wm1a, wm1b, wm2,
               o_ref, vbuf):
    f32 = jnp.float32
    feats = cf_ref[...]
    tw = tw_ref[...]
    tb = tb_ref[...]
    tsv = tsv_ref[...]
    twc = tw * _INV_2PI
    tbc = tb * _INV_2PI
    te0 = _cos_2pi(tbc)
    q = (jnp.dot(feats, wqa[...], preferred_element_type=f32)
         + jnp.dot(te0, wqb[...], preferred_element_type=f32))
    s_all = None
    for j in range(K):
        nbj = nbf_ref[:, j * D:(j + 1) * D]
        te = _cos_2pi((tsv - nbts_ref[:, j:j + 1]) * twc + tbc)
        efj = ef_ref[:, j:j + 1]
        kj = (jnp.dot(nbj, wka[...], preferred_element_type=f32)
              + jnp.dot(te, wkt[...], preferred_element_type=f32)
              + efj * wke[...])
        vj = (jnp.dot(nbj, wva[...], preferred_element_type=f32)
              + jnp.dot(te, wvt[...], preferred_element_type=f32)
              + efj * wve[...])
        vbuf[:, j * D:(j + 1) * D] = vj
        sj = jnp.dot(q * kj, sel_ref[j * D:(j + 1) * D, :],
                     preferred_element_type=f32)
        s_all = sj if s_all is None else s_all + sj
    inv = 1.0 / math.sqrt(D // 2)
    s0 = s_all[:, :K] * inv
    s1 = s_all[:, 16:16 + K] * inv
    e0 = jnp.exp(s0 - jnp.max(s0, axis=1, keepdims=True))
    e1 = jnp.exp(s1 - jnp.max(s1, axis=1, keepdims=True))
    a0 = e0 / jnp.sum(e0, axis=1, keepdims=True)
    a1 = e1 / jnp.sum(e1, axis=1, keepdims=True)
    out0 = a0[:, 0:1] * vbuf[:, 0:D // 2]
    out1 = a1[:, 0:1] * vbuf[:, D // 2:D]
    for j in range(1, K):
        out0 = out0 + a0[:, j:j + 1] * vbuf[:, j * D:j * D + D // 2]
        out1 = out1 + a1[:, j:j + 1] * vbuf[:, j * D + D // 2:(j + 1) * D]
    out = jnp.concatenate([out0, out1], axis=1)
    ao = jnp.dot(out, wo[...], preferred_element_type=f32)
    hid = jnp.maximum(jnp.dot(ao, wm1a[...], preferred_element_type=f32)
                      + jnp.dot(feats, wm1b[...], preferred_element_type=f32),
                      0.0)
    o_ref[...] = jnp.dot(hid, wm2[...], preferred_element_type=f32)


def _head_selector():
    sel = jnp.zeros((K * D, 32), jnp.float32)
    lane = jnp.arange(D)
    rows = []
    for j in range(K):
        col = jnp.where(lane < D // 2, j, 16 + j)
        rows.append(jax.nn.one_hot(col, 32, dtype=jnp.float32))
    return jnp.concatenate(rows, axis=0)


def _attn_layer(tsv, nbts, ef, cf, nbf, tw2, tb2, w, blk):
    n = cf.shape[0]
    sel = _head_selector()
    row = lambda width: pl.BlockSpec((blk, width), lambda b: (b, 0))
    full = lambda a: pl.BlockSpec(a.shape, lambda b: (0, 0))
    in_specs = [row(1), row(16), row(16), row(D), row(K * D),
                full(tw2), full(tb2), full(sel)] + [full(x) for x in w]
    return pl.pallas_call(
        _attn_body,
        grid=(n // blk,),
        in_specs=in_specs,
        out_specs=row(D),
        out_shape=jax.ShapeDtypeStruct((n, D), jnp.float32),
        scratch_shapes=[pltpu.VMEM((blk, K * D), jnp.float32)],
    )(tsv, nbts, ef, cf, nbf, tw2, tb2, sel, *w)


def _layer_weights(Wq, Wk, Wv, Wo, Wm1, Wm2, l):
    return (Wq[l, :D], Wq[l, D:],
            Wk[l, :D], Wk[l, D:D + 1], Wk[l, D + 1:],
            Wv[l, :D], Wv[l, D:D + 1], Wv[l, D + 1:],
            Wo[l], Wm1[l, :D], Wm1[l, D:], Wm2[l])


def kernel(source_nodes, destination_nodes, edge_times, edge_idxs, node_feat,
           edge_feat, memory, time_w, time_b, nbr_idx, nbr_eidx, nbr_ts,
           Wq, Wk, Wv, Wo, Wm1, Wm2):
    i32 = jnp.int32
    nodes = jnp.concatenate([source_nodes, destination_nodes]).astype(i32)
    ts2 = jnp.concatenate([edge_times, edge_times])

    nbrtab = jnp.concatenate(
        [nbr_idx.astype(i32), nbr_eidx.astype(i32),
         lax.bitcast_convert_type(nbr_ts, i32),
         jnp.zeros((N_NODES, 2), i32)], axis=1)

    feats = _feats_all(memory, node_feat)

    ta = _sc_gather(nbrtab, nodes, 64)
    c1idx = ta[:, :K].reshape(-1)
    eidx2 = ta[:, K:2 * K].reshape(-1)
    nbts2 = jnp.pad(lax.bitcast_convert_type(ta[:, 2 * K:3 * K], jnp.float32),
                    ((0, 0), (0, 16 - K)))

    tb_rows = _sc_gather(nbrtab, c1idx, 64)
    idx2 = tb_rows[:, :K].reshape(-1)
    eidx1 = tb_rows[:, K:2 * K].reshape(-1)
    nbts1 = jnp.pad(lax.bitcast_convert_type(tb_rows[:, 2 * K:3 * K],
                                             jnp.float32),
                    ((0, 0), (0, 16 - K)))

    ef_idx = jnp.concatenate([eidx1, eidx2])
    ef16 = jnp.pad(edge_feat, ((0, 0), (0, 15)))
    cf2, cf1, nbf, efg = _sc_gather_feats(feats, ef16, nodes, c1idx,
                                          idx2, ef_idx)
    ef1 = jnp.pad(efg[:N2, :1].reshape(N1, K), ((0, 0), (0, 16 - K)))
    ef2 = jnp.pad(efg[N2:, :1].reshape(B2, K), ((0, 0), (0, 16 - K)))

    tw2 = time_w.reshape(1, D)
    tb2 = time_b.reshape(1, D)
    tsr1 = jnp.repeat(ts2, K).reshape(N1, 1)
    w0 = _layer_weights(Wq, Wk, Wv, Wo, Wm1, Wm2, 0)
    w1 = _layer_weights(Wq, Wk, Wv, Wo, Wm1, Wm2, 1)

    l1 = _attn_layer(tsr1, nbts1, ef1, cf1, nbf.reshape(N1, K * D),
                     tw2, tb2, w0, 1024)
    out = _attn_layer(ts2.reshape(B2, 1), nbts2, ef2, cf2,
                      l1.reshape(B2, K * D), tw2, tb2, w1, 1024)
    nS = source_nodes.shape[0]
    return (out[:nS], out[nS:])
